# baseline (speedup 1.0000x reference)
import jax
import jax.numpy as jnp
from jax.experimental import pallas as pl
from jax.experimental.pallas import tpu as pltpu
from jax.experimental.pallas import tpu_sc as plsc

B = 4096
IN_DIM = 128
OUT_DIM = 256
N_BINS = 50
TAU = 10.0

NCORES = 2
NSUB = 16
NW = NCORES * NSUB
ROWS_PER = B // NW
LAG = 8


def _sc_probe(lat_hbm, spk_hbm, zbuf, sem):
    c = jax.lax.axis_index("core")
    s = jax.lax.axis_index("subcore")
    base = (c * NSUB + s) * ROWS_PER

    @pl.loop(0, N_BINS)
    def _(i):
        @pl.loop(0, OUT_DIM, step=16)
        def _(j):
            zbuf.at[pl.ds(i, 1), pl.ds(j, 16)][...] = jnp.zeros(
                (1, 16), jnp.float32)

    def cp(r):
        return pltpu.make_async_copy(zbuf, spk_hbm.at[base + r], sem)

    @pl.loop(0, LAG)
    def _(r):
        cp(r).start()

    @pl.loop(LAG, ROWS_PER)
    def _(r):
        cp(r).start()
        cp(r - LAG).wait()

    @pl.loop(0, LAG)
    def _(j):
        cp(ROWS_PER - LAG + j).wait()


def kernel(x, W, b):
    mesh = plsc.VectorSubcoreMesh(core_axis_name="core",
                                  subcore_axis_name="subcore")
    lat, spikes = pl.kernel(
        _sc_probe,
        out_type=(jax.ShapeDtypeStruct((B, OUT_DIM), jnp.int32),
                  jax.ShapeDtypeStruct((B, N_BINS, OUT_DIM), jnp.float32)),
        mesh=mesh,
        scratch_types=[pltpu.VMEM((N_BINS, OUT_DIM), jnp.float32),
                       pltpu.SemaphoreType.DMA],
    )()
    return (lat, spikes)


# speedup vs baseline: 1.0188x; 1.0188x over previous
import jax
import jax.numpy as jnp
from jax.experimental import pallas as pl
from jax.experimental.pallas import tpu as pltpu
from jax.experimental.pallas import tpu_sc as plsc

B = 4096
IN_DIM = 128
OUT_DIM = 256
N_BINS = 50
TAU = 10.0

S = 2048
CH = 64
NBUF = 8
NCHUNKS_TC = S // CH

NCORES = 2
NSUB = 16
NW = NCORES * NSUB
SC_ROWS = (B - S) // NW
LAG = 8


def _tc_half(x_ref, wt_ref, b_ref, lat_hbm, spk_hbm,
             spk_buf, lat_buf, spk_sem, lat_sem):
    bins = jax.lax.broadcasted_iota(jnp.int32, (CH, N_BINS, OUT_DIM), 1)

    def spk_copy(i, slot):
        return pltpu.make_async_copy(
            spk_buf.at[slot], spk_hbm.at[pl.ds(i * CH, CH)], spk_sem.at[slot])

    def lat_copy(i, slot):
        return pltpu.make_async_copy(
            lat_buf.at[slot], lat_hbm.at[pl.ds(i * CH, CH)], lat_sem.at[slot])

    def body(i, carry):
        slot = jax.lax.rem(i, NBUF)

        @pl.when(i >= NBUF)
        def _():
            spk_copy(i - NBUF, slot).wait()
            lat_copy(i - NBUF, slot).wait()

        xs = x_ref[pl.ds(i * CH, CH), :]
        rates = jax.lax.dot_general(
            xs, wt_ref[...],
            dimension_numbers=(((1,), (0,)), ((), ())),
            preferred_element_type=jnp.float32,
        ) + b_ref[...]
        lat = jnp.clip(N_BINS * jnp.exp(-rates / TAU), 1, N_BINS - 1
                       ).astype(jnp.int32)
        lat_buf[slot] = lat
        spk_buf[slot] = (bins == lat[:, None, :]).astype(jnp.float32)

        spk_copy(i, slot).start()
        lat_copy(i, slot).start()
        return carry

    jax.lax.fori_loop(0, NCHUNKS_TC, body, 0)

    def drain(j, carry):
        i = NCHUNKS_TC - NBUF + j
        slot = jax.lax.rem(i, NBUF)
        spk_copy(i, slot).wait()
        lat_copy(i, slot).wait()
        return carry

    jax.lax.fori_loop(0, NBUF, drain, 0)


def _sc_half(spk_hbm, zbuf, sem):
    c = jax.lax.axis_index("core")
    s = jax.lax.axis_index("subcore")
    base = S + (c * NSUB + s) * SC_ROWS

    @pl.loop(0, N_BINS)
    def _(i):
        @pl.loop(0, OUT_DIM, step=16)
        def _(j):
            zbuf.at[pl.ds(i, 1), pl.ds(j, 16)][...] = jnp.zeros(
                (1, 16), jnp.float32)

    def cp(r):
        return pltpu.make_async_copy(zbuf, spk_hbm.at[base + r], sem)

    @pl.loop(0, LAG)
    def _(r):
        cp(r).start()

    @pl.loop(LAG, SC_ROWS)
    def _(r):
        cp(r).start()
        cp(r - LAG).wait()

    @pl.loop(0, LAG)
    def _(j):
        cp(SC_ROWS - LAG + j).wait()


def kernel(x, W, b):
    wt = W.T
    b2 = b.reshape(1, OUT_DIM)

    lat, _spk_a = pl.pallas_call(
        _tc_half,
        in_specs=[
            pl.BlockSpec(memory_space=pltpu.MemorySpace.VMEM),
            pl.BlockSpec(memory_space=pltpu.MemorySpace.VMEM),
            pl.BlockSpec(memory_space=pltpu.MemorySpace.VMEM),
        ],
        out_specs=[
            pl.BlockSpec(memory_space=pltpu.MemorySpace.HBM),
            pl.BlockSpec(memory_space=pltpu.MemorySpace.HBM),
        ],
        out_shape=[
            jax.ShapeDtypeStruct((B, OUT_DIM), jnp.int32),
            jax.ShapeDtypeStruct((B, N_BINS, OUT_DIM), jnp.float32),
        ],
        scratch_shapes=[
            pltpu.VMEM((NBUF, CH, N_BINS, OUT_DIM), jnp.float32),
            pltpu.VMEM((NBUF, CH, OUT_DIM), jnp.int32),
            pltpu.SemaphoreType.DMA((NBUF,)),
            pltpu.SemaphoreType.DMA((NBUF,)),
        ],
    )(x, wt, b2)

    mesh = plsc.VectorSubcoreMesh(core_axis_name="core",
                                  subcore_axis_name="subcore")
    spikes = pl.kernel(
        _sc_half,
        out_type=jax.ShapeDtypeStruct((B, N_BINS, OUT_DIM), jnp.float32),
        mesh=mesh,
        scratch_types=[pltpu.VMEM((N_BINS, OUT_DIM), jnp.float32),
                       pltpu.SemaphoreType.DMA],
    )()
    return (lat, spikes)


# speedup vs baseline: 1.0193x; 1.0005x over previous
import jax
import jax.numpy as jnp
from jax.experimental import pallas as pl
from jax.experimental.pallas import tpu as pltpu
from jax.experimental.pallas import tpu_sc as plsc

B = 4096
IN_DIM = 128
OUT_DIM = 256
N_BINS = 50
TAU = 10.0

S = 2048
CH = 64
NBUF = 8
NCHUNKS_TC = S // CH

NCORES = 2
NSUB = 16
NW = NCORES * NSUB
SC_ROWS = (B - S) // NW
LAG = 8


def _tc_half(x_ref, wt_ref, b_ref, lat_hbm, spk_hbm,
             spk_buf, lat_buf, spk_sem, lat_sem):
    bins = jax.lax.broadcasted_iota(jnp.int32, (CH, N_BINS, OUT_DIM), 1)

    def spk_copy(i, slot):
        return pltpu.make_async_copy(
            spk_buf.at[slot], spk_hbm.at[pl.ds(i * CH, CH)], spk_sem.at[slot])

    def lat_copy(i, slot):
        return pltpu.make_async_copy(
            lat_buf.at[slot], lat_hbm.at[pl.ds(i * CH, CH)], lat_sem.at[slot])

    def body(i, carry):
        slot = jax.lax.rem(i, NBUF)

        @pl.when(i >= NBUF)
        def _():
            spk_copy(i - NBUF, slot).wait()
            lat_copy(i - NBUF, slot).wait()

        xs = x_ref[pl.ds(i * CH, CH), :]
        rates = jax.lax.dot_general(
            xs, wt_ref[...],
            dimension_numbers=(((1,), (0,)), ((), ())),
            preferred_element_type=jnp.float32,
        ) + b_ref[...]
        lat = jnp.clip(N_BINS * jnp.exp(-rates / TAU), 1, N_BINS - 1
                       ).astype(jnp.int32)
        lat_buf[slot] = lat
        spk_buf[slot] = (bins == lat[:, None, :]).astype(jnp.float32)

        spk_copy(i, slot).start()
        lat_copy(i, slot).start()
        return carry

    jax.lax.fori_loop(0, NCHUNKS_TC, body, 0)

    def drain(j, carry):
        i = NCHUNKS_TC - NBUF + j
        slot = jax.lax.rem(i, NBUF)
        spk_copy(i, slot).wait()
        lat_copy(i, slot).wait()
        return carry

    jax.lax.fori_loop(0, NBUF, drain, 0)


def _sc_half(spk_hbm, zbuf, sem):
    c = jax.lax.axis_index("core")
    s = jax.lax.axis_index("subcore")
    base = S + (c * NSUB + s) * SC_ROWS

    @pl.loop(0, N_BINS)
    def _(i):
        @pl.loop(0, OUT_DIM, step=16)
        def _(j):
            zbuf.at[pl.ds(i, 1), pl.ds(j, 16)][...] = jnp.zeros(
                (1, 16), jnp.float32)

    def cp(r):
        return pltpu.make_async_copy(zbuf, spk_hbm.at[base + r], sem)

    @pl.loop(0, LAG)
    def _(r):
        cp(r).start()

    @pl.loop(LAG, SC_ROWS)
    def _(r):
        cp(r).start()
        cp(r - LAG).wait()

    @pl.loop(0, LAG)
    def _(j):
        cp(SC_ROWS - LAG + j).wait()


def kernel(x, W, b):
    wt = W.T
    b2 = b.reshape(1, OUT_DIM)

    mesh = plsc.VectorSubcoreMesh(core_axis_name="core",
                                  subcore_axis_name="subcore")
    spikes = pl.kernel(
        _sc_half,
        out_type=jax.ShapeDtypeStruct((B, N_BINS, OUT_DIM), jnp.float32),
        mesh=mesh,
        scratch_types=[pltpu.VMEM((N_BINS, OUT_DIM), jnp.float32),
                       pltpu.SemaphoreType.DMA],
    )()

    lat, _spk_a = pl.pallas_call(
        _tc_half,
        in_specs=[
            pl.BlockSpec(memory_space=pltpu.MemorySpace.VMEM),
            pl.BlockSpec(memory_space=pltpu.MemorySpace.VMEM),
            pl.BlockSpec(memory_space=pltpu.MemorySpace.VMEM),
        ],
        out_specs=[
            pl.BlockSpec(memory_space=pltpu.MemorySpace.HBM),
            pl.BlockSpec(memory_space=pltpu.MemorySpace.HBM),
        ],
        out_shape=[
            jax.ShapeDtypeStruct((B, OUT_DIM), jnp.int32),
            jax.ShapeDtypeStruct((B, N_BINS, OUT_DIM), jnp.float32),
        ],
        scratch_shapes=[
            pltpu.VMEM((NBUF, CH, N_BINS, OUT_DIM), jnp.float32),
            pltpu.VMEM((NBUF, CH, OUT_DIM), jnp.int32),
            pltpu.SemaphoreType.DMA((NBUF,)),
            pltpu.SemaphoreType.DMA((NBUF,)),
        ],
    )(x, wt, b2)
    return (lat, spikes)


# speedup vs baseline: 1.0989x; 1.0781x over previous
import jax
import jax.numpy as jnp
from jax.experimental import pallas as pl

B = 4096
IN_DIM = 128
OUT_DIM = 256
N_BINS = 50
TAU = 10.0

BS = 128


def _encoder_block(x_ref, w_ref, b_ref, lat_ref, spk_ref):
    rates = jax.lax.dot_general(
        x_ref[...], w_ref[...],
        dimension_numbers=(((1,), (1,)), ((), ())),
        preferred_element_type=jnp.float32,
    ) + b_ref[...]
    lat = jnp.clip(N_BINS * jnp.exp(-rates / TAU), 1, N_BINS - 1).astype(jnp.int32)
    lat_ref[...] = lat
    bins = jax.lax.broadcasted_iota(jnp.int32, (BS, N_BINS, OUT_DIM), 1)
    spk_ref[...] = (bins == lat[:, None, :]).astype(jnp.float32)


def kernel(x, W, b):
    b2 = b.reshape(1, OUT_DIM)
    grid = (B // BS,)
    lat, spikes = pl.pallas_call(
        _encoder_block,
        grid=grid,
        in_specs=[
            pl.BlockSpec((BS, IN_DIM), lambda i: (i, 0)),
            pl.BlockSpec((OUT_DIM, IN_DIM), lambda i: (0, 0)),
            pl.BlockSpec((1, OUT_DIM), lambda i: (0, 0)),
        ],
        out_specs=[
            pl.BlockSpec((BS, OUT_DIM), lambda i: (i, 0)),
            pl.BlockSpec((BS, N_BINS, OUT_DIM), lambda i: (i, 0, 0)),
        ],
        out_shape=[
            jax.ShapeDtypeStruct((B, OUT_DIM), jnp.int32),
            jax.ShapeDtypeStruct((B, N_BINS, OUT_DIM), jnp.float32),
        ],
    )(x, W, b2)
    return (lat, spikes)
